# baseline (speedup 1.0000x reference)
import functools

import jax
import jax.numpy as jnp
from jax import lax
from jax.experimental import pallas as pl
from jax.experimental.pallas import tpu as pltpu
from jax.experimental.pallas import tpu_sc as plsc

N = 160000
BN = 6400
NB = N // BN
NUM = 16 ** 4
CH = 8000
NCHUNK = N // CH
NWORK = 32
RPW = 256 // NWORK
BV = 4096
NBV = NUM // BV
NEG_INF = float("-inf")


def _hsw(x):
    return x * jnp.clip(x + 3.0, 0.0, 6.0) * (1.0 / 6.0)


def _dot(a, b):
    return lax.dot_general(a, b, (((1,), (0,)), ((), ())),
                           preferred_element_type=jnp.float32,
                           precision=lax.Precision.HIGHEST)


def _ka_body(ct_ref, ft_ref, ids_ref, moms_ref, acc_ref):
    i = pl.program_id(0)

    @pl.when(i == 0)
    def _():
        acc_ref[...] = jnp.zeros_like(acc_ref)

    c = ct_ref[...]
    ids_ref[...] = ((c[0:1] * 16 + c[1:2]) * 16 + c[2:3]) * 16 + c[3:4]
    f = ft_ref[...]
    acc_ref[0:4, 0:1] += jnp.sum(f, axis=1, keepdims=True)
    acc_ref[4:8, 0:1] += jnp.sum(f * f, axis=1, keepdims=True)
    moms_ref[...] = acc_ref[...]


_ka = pl.pallas_call(
    _ka_body,
    grid=(NB,),
    in_specs=[pl.BlockSpec((4, BN), lambda i: (0, i)),
              pl.BlockSpec((4, BN), lambda i: (0, i))],
    out_specs=[pl.BlockSpec((1, BN), lambda i: (0, i)),
               pl.BlockSpec((8, 128), lambda i: (0, 0))],
    out_shape=[jax.ShapeDtypeStruct((1, N), jnp.int32),
               jax.ShapeDtypeStruct((8, 128), jnp.float32)],
    scratch_shapes=[pltpu.VMEM((8, 128), jnp.float32)],
)


def _kb_body(ft_ref, m4_ref, s4_ref, b4_ref, w0_ref, moms_ref, acc_ref):
    i = pl.program_id(0)

    @pl.when(i == 0)
    def _():
        acc_ref[...] = jnp.zeros_like(acc_ref)

    x = (ft_ref[...] - m4_ref[...]) * s4_ref[...] + b4_ref[...]
    h0 = _dot(w0_ref[...], x)
    acc_ref[:, 0:1] += jnp.sum(h0, axis=1, keepdims=True)
    acc_ref[:, 1:2] += jnp.sum(h0 * h0, axis=1, keepdims=True)
    moms_ref[...] = acc_ref[...]


_kb = pl.pallas_call(
    _kb_body,
    grid=(NB,),
    in_specs=[pl.BlockSpec((4, BN), lambda i: (0, i)),
              pl.BlockSpec((4, 1), lambda i: (0, 0)),
              pl.BlockSpec((4, 1), lambda i: (0, 0)),
              pl.BlockSpec((4, 1), lambda i: (0, 0)),
              pl.BlockSpec((64, 4), lambda i: (0, 0))],
    out_specs=pl.BlockSpec((64, 128), lambda i: (0, 0)),
    out_shape=jax.ShapeDtypeStruct((64, 128), jnp.float32),
    scratch_shapes=[pltpu.VMEM((64, 128), jnp.float32)],
)


def _kc_body(ft_ref, m4_ref, s4_ref, b4_ref, w0_ref, s0_ref, t0_ref, w1_ref,
             moms_ref, acc_ref):
    i = pl.program_id(0)

    @pl.when(i == 0)
    def _():
        acc_ref[...] = jnp.zeros_like(acc_ref)

    x = (ft_ref[...] - m4_ref[...]) * s4_ref[...] + b4_ref[...]
    h0 = _dot(w0_ref[...], x)
    u = _hsw(h0 * s0_ref[...] + t0_ref[...])
    h1 = _dot(w1_ref[...], u)
    acc_ref[:, 0:1] += jnp.sum(h1, axis=1, keepdims=True)
    acc_ref[:, 1:2] += jnp.sum(h1 * h1, axis=1, keepdims=True)
    moms_ref[...] = acc_ref[...]


_kc = pl.pallas_call(
    _kc_body,
    grid=(NB,),
    in_specs=[pl.BlockSpec((4, BN), lambda i: (0, i)),
              pl.BlockSpec((4, 1), lambda i: (0, 0)),
              pl.BlockSpec((4, 1), lambda i: (0, 0)),
              pl.BlockSpec((4, 1), lambda i: (0, 0)),
              pl.BlockSpec((64, 4), lambda i: (0, 0)),
              pl.BlockSpec((64, 1), lambda i: (0, 0)),
              pl.BlockSpec((64, 1), lambda i: (0, 0)),
              pl.BlockSpec((128, 64), lambda i: (0, 0))],
    out_specs=pl.BlockSpec((128, 128), lambda i: (0, 0)),
    out_shape=jax.ShapeDtypeStruct((128, 128), jnp.float32),
    scratch_shapes=[pltpu.VMEM((128, 128), jnp.float32)],
)


def _kd_body(ft_ref, m4_ref, s4_ref, b4_ref, w0_ref, s0_ref, t0_ref, w1_ref,
             s1_ref, t1_ref, w2_ref, b2_ref, ht_ref):
    x = (ft_ref[...] - m4_ref[...]) * s4_ref[...] + b4_ref[...]
    h0 = _dot(w0_ref[...], x)
    u = _hsw(h0 * s0_ref[...] + t0_ref[...])
    h1 = _dot(w1_ref[...], u)
    v = _hsw(h1 * s1_ref[...] + t1_ref[...])
    ht_ref[...] = _dot(w2_ref[...], v) + b2_ref[...]


_kd = pl.pallas_call(
    _kd_body,
    grid=(NB,),
    in_specs=[pl.BlockSpec((4, BN), lambda i: (0, i)),
              pl.BlockSpec((4, 1), lambda i: (0, 0)),
              pl.BlockSpec((4, 1), lambda i: (0, 0)),
              pl.BlockSpec((4, 1), lambda i: (0, 0)),
              pl.BlockSpec((64, 4), lambda i: (0, 0)),
              pl.BlockSpec((64, 1), lambda i: (0, 0)),
              pl.BlockSpec((64, 1), lambda i: (0, 0)),
              pl.BlockSpec((128, 64), lambda i: (0, 0)),
              pl.BlockSpec((128, 1), lambda i: (0, 0)),
              pl.BlockSpec((128, 1), lambda i: (0, 0)),
              pl.BlockSpec((256, 128), lambda i: (0, 0)),
              pl.BlockSpec((256, 1), lambda i: (0, 0))],
    out_specs=pl.BlockSpec((256, BN), lambda i: (0, i)),
    out_shape=jax.ShapeDtypeStruct((256, N), jnp.float32),
    scratch_shapes=[],
)


def _perm16(x, idx):
    dn = lax.GatherDimensionNumbers(offset_dims=(), collapsed_slice_dims=(0,),
                                    start_index_map=(0,))
    return lax.gather(x, idx[:, None], dn, (1,),
                      mode=lax.GatherScatterMode.PROMISE_IN_BOUNDS)


def _seg_body(ids_hbm, ht_hbm, ninf_hbm, out_hbm, acc, idb, valb):
    cid = lax.axis_index("c")
    sid = lax.axis_index("s")
    wid = sid * 2 + cid

    lanes = lax.iota(jnp.int32, 16)
    zero = jnp.zeros((16,), jnp.int32)
    sh1 = jnp.maximum(lanes - 1, zero)
    sh2 = jnp.maximum(lanes - 2, zero)
    sh4 = jnp.maximum(lanes - 4, zero)
    sh8 = jnp.maximum(lanes - 8, zero)
    dn1 = jnp.minimum(lanes + 1, zero + 15)
    is15 = lanes == (zero + 15)

    def round_body(r, carry):
        ch = wid * RPW + r
        pltpu.sync_copy(ninf_hbm, acc)

        def chunk_body(k, carry2):
            kk = lax.rem(k + wid, NCHUNK)
            base = kk * CH
            pltpu.sync_copy(ids_hbm.at[pl.ds(base, CH)], idb)
            pltpu.sync_copy(ht_hbm.at[pl.ds(ch * N + base, CH)], valb)

            def vec_body(j, carry3):
                for jj in range(2):
                    o = (j * 2 + jj) * 16
                    idv = idb[pl.ds(o, 16)]
                    val = valb[pl.ds(o, 16)]
                    cur0 = plsc.load_gather(acc, [idv])
                    sk, perm = plsc.sort_key_val(idv, lanes)
                    sv = _perm16(val, perm)
                    for sh in (sh1, sh2, sh4, sh8):
                        pk = _perm16(sk, sh)
                        pv = _perm16(sv, sh)
                        sv = jnp.maximum(sv, jnp.where(pk == sk, pv, NEG_INF))
                    last = (sk != _perm16(sk, dn1)) | is15
                    cur = _perm16(cur0, perm)
                    m = last & (sv > cur)
                    plsc.store_scatter(acc, [sk], sv, mask=m)
                return carry3

            lax.fori_loop(0, CH // 32, vec_body, 0)
            return carry2

        lax.fori_loop(0, NCHUNK, chunk_body, 0)
        pltpu.sync_copy(acc, out_hbm.at[pl.ds(ch * NUM, NUM)])
        return carry

    lax.fori_loop(0, RPW, round_body, 0)


_seg = pl.kernel(
    _seg_body,
    out_type=jax.ShapeDtypeStruct((256 * NUM,), jnp.float32),
    scratch_types=[pltpu.VMEM((NUM,), jnp.float32),
                   pltpu.VMEM((CH,), jnp.int32),
                   pltpu.VMEM((CH,), jnp.float32)],
    mesh=plsc.VectorSubcoreMesh(core_axis_name="c", subcore_axis_name="s"),
    compiler_params=pltpu.CompilerParams(needs_layout_passes=False),
)


def _ke_body(vm_ref, wc_ref, bc_ref, vf_ref):
    vb = vm_ref[...]
    occ = vb[0:1, :] > NEG_INF
    vm = jnp.where(occ, vb, 0.0)
    z = _dot(wc_ref[...], vm) + bc_ref[...]
    vf_ref[...] = jnp.where(occ, _hsw(z), 0.0)


_ke = pl.pallas_call(
    _ke_body,
    grid=(NBV,),
    in_specs=[pl.BlockSpec((256, BV), lambda i: (0, i)),
              pl.BlockSpec((16, 256), lambda i: (0, 0)),
              pl.BlockSpec((16, 1), lambda i: (0, 0))],
    out_specs=pl.BlockSpec((16, BV), lambda i: (0, i)),
    out_shape=jax.ShapeDtypeStruct((16, NUM), jnp.float32),
    scratch_shapes=[],
)


def kernel(features, coors, pre_g, pre_b, W0, g0, b0, W1, g1, b1, W2, b2, Wc, bc):
    fT = features.T
    cT = coors.T
    ids2d, momsA = _ka(cT, fT)
    ids = ids2d.reshape(N)

    sA = momsA[0:4, 0]
    qA = momsA[4:8, 0]
    m = sA / N
    var = qA / N - m * m
    rstd = lax.rsqrt(var + 1e-5)
    m4 = m[:, None]
    s4 = (pre_g * rstd)[:, None]
    b4 = pre_b[:, None]
    W0t = W0.T
    W1t = W1.T
    W2t = W2.T
    Wct = Wc.T

    momsB = _kb(fT, m4, s4, b4, W0t)
    m0 = momsB[:, 0] / N
    v0 = momsB[:, 1] / N - m0 * m0
    r0 = lax.rsqrt(v0 + 1e-5)
    s0 = (g0 * r0)[:, None]
    t0 = (b0 - m0 * g0 * r0)[:, None]

    momsC = _kc(fT, m4, s4, b4, W0t, s0, t0, W1t)
    m1 = momsC[:, 0] / N
    v1 = momsC[:, 1] / N - m1 * m1
    r1 = lax.rsqrt(v1 + 1e-5)
    s1 = (g1 * r1)[:, None]
    t1 = (b1 - m1 * g1 * r1)[:, None]

    hT = _kd(fT, m4, s4, b4, W0t, s0, t0, W1t, s1, t1, W2t, b2[:, None])

    ninf = jnp.full((NUM,), NEG_INF, jnp.float32)
    vmaxT = _seg(ids, hT.reshape(256 * N), ninf).reshape(256, NUM)

    vfT = _ke(vmaxT, Wct, bc[:, None])

    t5 = vfT.reshape(16, 16, 16, 16, 16)
    t = jnp.transpose(t5, (1, 0, 4, 3, 2))
    return jnp.pad(t, ((0, 0), (0, 0), (0, 0), (0, 48), (0, 48)))


# speedup vs baseline: 1.0519x; 1.0519x over previous
import functools

import jax
import jax.numpy as jnp
from jax import lax
from jax.experimental import pallas as pl
from jax.experimental.pallas import tpu as pltpu
from jax.experimental.pallas import tpu_sc as plsc

N = 160000
BN = 6400
NB = N // BN
NUM = 16 ** 4
CH = 20000
NCHUNK = N // CH
NWORK = 32
RPW = 256 // NWORK
BV = 4096
NBV = NUM // BV
NEG_INF = float("-inf")


def _hsw(x):
    return x * jnp.clip(x + 3.0, 0.0, 6.0) * (1.0 / 6.0)


def _dot(a, b):
    return lax.dot_general(a, b, (((1,), (0,)), ((), ())),
                           preferred_element_type=jnp.float32,
                           precision=lax.Precision.HIGHEST)


def _ka_body(ct_ref, ft_ref, ids_ref, moms_ref, acc_ref):
    i = pl.program_id(0)

    @pl.when(i == 0)
    def _():
        acc_ref[...] = jnp.zeros_like(acc_ref)

    c = ct_ref[...]
    ids_ref[...] = ((c[0:1] * 16 + c[1:2]) * 16 + c[2:3]) * 16 + c[3:4]
    f = ft_ref[...]
    acc_ref[0:4, 0:1] += jnp.sum(f, axis=1, keepdims=True)
    acc_ref[4:8, 0:1] += jnp.sum(f * f, axis=1, keepdims=True)
    moms_ref[...] = acc_ref[...]


_ka = pl.pallas_call(
    _ka_body,
    grid=(NB,),
    in_specs=[pl.BlockSpec((4, BN), lambda i: (0, i)),
              pl.BlockSpec((4, BN), lambda i: (0, i))],
    out_specs=[pl.BlockSpec((1, BN), lambda i: (0, i)),
               pl.BlockSpec((8, 128), lambda i: (0, 0))],
    out_shape=[jax.ShapeDtypeStruct((1, N), jnp.int32),
               jax.ShapeDtypeStruct((8, 128), jnp.float32)],
    scratch_shapes=[pltpu.VMEM((8, 128), jnp.float32)],
)


def _kb_body(ft_ref, m4_ref, s4_ref, b4_ref, w0_ref, moms_ref, acc_ref):
    i = pl.program_id(0)

    @pl.when(i == 0)
    def _():
        acc_ref[...] = jnp.zeros_like(acc_ref)

    x = (ft_ref[...] - m4_ref[...]) * s4_ref[...] + b4_ref[...]
    h0 = _dot(w0_ref[...], x)
    acc_ref[:, 0:1] += jnp.sum(h0, axis=1, keepdims=True)
    acc_ref[:, 1:2] += jnp.sum(h0 * h0, axis=1, keepdims=True)
    moms_ref[...] = acc_ref[...]


_kb = pl.pallas_call(
    _kb_body,
    grid=(NB,),
    in_specs=[pl.BlockSpec((4, BN), lambda i: (0, i)),
              pl.BlockSpec((4, 1), lambda i: (0, 0)),
              pl.BlockSpec((4, 1), lambda i: (0, 0)),
              pl.BlockSpec((4, 1), lambda i: (0, 0)),
              pl.BlockSpec((64, 4), lambda i: (0, 0))],
    out_specs=pl.BlockSpec((64, 128), lambda i: (0, 0)),
    out_shape=jax.ShapeDtypeStruct((64, 128), jnp.float32),
    scratch_shapes=[pltpu.VMEM((64, 128), jnp.float32)],
)


def _kc_body(ft_ref, m4_ref, s4_ref, b4_ref, w0_ref, s0_ref, t0_ref, w1_ref,
             moms_ref, acc_ref):
    i = pl.program_id(0)

    @pl.when(i == 0)
    def _():
        acc_ref[...] = jnp.zeros_like(acc_ref)

    x = (ft_ref[...] - m4_ref[...]) * s4_ref[...] + b4_ref[...]
    h0 = _dot(w0_ref[...], x)
    u = _hsw(h0 * s0_ref[...] + t0_ref[...])
    h1 = _dot(w1_ref[...], u)
    acc_ref[:, 0:1] += jnp.sum(h1, axis=1, keepdims=True)
    acc_ref[:, 1:2] += jnp.sum(h1 * h1, axis=1, keepdims=True)
    moms_ref[...] = acc_ref[...]


_kc = pl.pallas_call(
    _kc_body,
    grid=(NB,),
    in_specs=[pl.BlockSpec((4, BN), lambda i: (0, i)),
              pl.BlockSpec((4, 1), lambda i: (0, 0)),
              pl.BlockSpec((4, 1), lambda i: (0, 0)),
              pl.BlockSpec((4, 1), lambda i: (0, 0)),
              pl.BlockSpec((64, 4), lambda i: (0, 0)),
              pl.BlockSpec((64, 1), lambda i: (0, 0)),
              pl.BlockSpec((64, 1), lambda i: (0, 0)),
              pl.BlockSpec((128, 64), lambda i: (0, 0))],
    out_specs=pl.BlockSpec((128, 128), lambda i: (0, 0)),
    out_shape=jax.ShapeDtypeStruct((128, 128), jnp.float32),
    scratch_shapes=[pltpu.VMEM((128, 128), jnp.float32)],
)


def _kd_body(ft_ref, m4_ref, s4_ref, b4_ref, w0_ref, s0_ref, t0_ref, w1_ref,
             s1_ref, t1_ref, w2_ref, b2_ref, ht_ref):
    x = (ft_ref[...] - m4_ref[...]) * s4_ref[...] + b4_ref[...]
    h0 = _dot(w0_ref[...], x)
    u = _hsw(h0 * s0_ref[...] + t0_ref[...])
    h1 = _dot(w1_ref[...], u)
    v = _hsw(h1 * s1_ref[...] + t1_ref[...])
    ht_ref[...] = _dot(w2_ref[...], v) + b2_ref[...]


_kd = pl.pallas_call(
    _kd_body,
    grid=(NB,),
    in_specs=[pl.BlockSpec((4, BN), lambda i: (0, i)),
              pl.BlockSpec((4, 1), lambda i: (0, 0)),
              pl.BlockSpec((4, 1), lambda i: (0, 0)),
              pl.BlockSpec((4, 1), lambda i: (0, 0)),
              pl.BlockSpec((64, 4), lambda i: (0, 0)),
              pl.BlockSpec((64, 1), lambda i: (0, 0)),
              pl.BlockSpec((64, 1), lambda i: (0, 0)),
              pl.BlockSpec((128, 64), lambda i: (0, 0)),
              pl.BlockSpec((128, 1), lambda i: (0, 0)),
              pl.BlockSpec((128, 1), lambda i: (0, 0)),
              pl.BlockSpec((256, 128), lambda i: (0, 0)),
              pl.BlockSpec((256, 1), lambda i: (0, 0))],
    out_specs=pl.BlockSpec((256, BN), lambda i: (0, i)),
    out_shape=jax.ShapeDtypeStruct((256, N), jnp.float32),
    scratch_shapes=[],
)


def _perm16(x, idx):
    dn = lax.GatherDimensionNumbers(offset_dims=(), collapsed_slice_dims=(0,),
                                    start_index_map=(0,))
    return lax.gather(x, idx[:, None], dn, (1,),
                      mode=lax.GatherScatterMode.PROMISE_IN_BOUNDS)


def _seg_body(ids_hbm, ht_hbm, ninf_hbm, out_hbm, acc, idb, valb):
    cid = lax.axis_index("c")
    sid = lax.axis_index("s")
    wid = sid * 2 + cid

    lanes = lax.iota(jnp.int32, 16)
    zero = jnp.zeros((16,), jnp.int32)
    sh1 = jnp.maximum(lanes - 1, zero)
    sh2 = jnp.maximum(lanes - 2, zero)
    sh4 = jnp.maximum(lanes - 4, zero)
    sh8 = jnp.maximum(lanes - 8, zero)
    dn1 = jnp.minimum(lanes + 1, zero + 15)
    is15 = lanes == (zero + 15)

    def round_body(r, carry):
        ch = wid * RPW + r
        pltpu.sync_copy(ninf_hbm, acc)

        def chunk_body(k, carry2):
            kk = lax.rem(k + wid, NCHUNK)
            base = kk * CH
            pltpu.sync_copy(ids_hbm.at[pl.ds(base, CH)], idb)
            pltpu.sync_copy(ht_hbm.at[pl.ds(ch * N + base, CH)], valb)

            def vec_body(j, carry3):
                idv = idb[pl.ds(j * 16, 16)]
                val = valb[pl.ds(j * 16, 16)]
                sk, sv = plsc.sort_key_val(idv, val)
                for sh in (sh1, sh2, sh4, sh8):
                    pk = _perm16(sk, sh)
                    pv = _perm16(sv, sh)
                    sv = jnp.maximum(sv, jnp.where(pk == sk, pv, NEG_INF))
                last = (sk != _perm16(sk, dn1)) | is15
                cur = plsc.load_gather(acc, [sk])
                m = last & (sv > cur)
                plsc.store_scatter(acc, [sk], sv, mask=m)
                return carry3

            lax.fori_loop(0, CH // 16, vec_body, 0)
            return carry2

        lax.fori_loop(0, NCHUNK, chunk_body, 0)
        pltpu.sync_copy(acc, out_hbm.at[pl.ds(ch * NUM, NUM)])
        return carry

    lax.fori_loop(0, RPW, round_body, 0)


_seg = pl.kernel(
    _seg_body,
    out_type=jax.ShapeDtypeStruct((256 * NUM,), jnp.float32),
    scratch_types=[pltpu.VMEM((NUM,), jnp.float32),
                   pltpu.VMEM((CH,), jnp.int32),
                   pltpu.VMEM((CH,), jnp.float32)],
    mesh=plsc.VectorSubcoreMesh(core_axis_name="c", subcore_axis_name="s"),
    compiler_params=pltpu.CompilerParams(needs_layout_passes=False),
)


def _ke_body(vm_ref, wc_ref, bc_ref, vf_ref):
    vb = vm_ref[...]
    occ = vb[0:1, :] > NEG_INF
    vm = jnp.where(occ, vb, 0.0)
    z = _dot(wc_ref[...], vm) + bc_ref[...]
    vf_ref[...] = jnp.where(occ, _hsw(z), 0.0)


_ke = pl.pallas_call(
    _ke_body,
    grid=(NBV,),
    in_specs=[pl.BlockSpec((256, BV), lambda i: (0, i)),
              pl.BlockSpec((16, 256), lambda i: (0, 0)),
              pl.BlockSpec((16, 1), lambda i: (0, 0))],
    out_specs=pl.BlockSpec((16, BV), lambda i: (0, i)),
    out_shape=jax.ShapeDtypeStruct((16, NUM), jnp.float32),
    scratch_shapes=[],
)


def kernel(features, coors, pre_g, pre_b, W0, g0, b0, W1, g1, b1, W2, b2, Wc, bc):
    fT = features.T
    cT = coors.T
    ids2d, momsA = _ka(cT, fT)
    ids = ids2d.reshape(N)

    sA = momsA[0:4, 0]
    qA = momsA[4:8, 0]
    m = sA / N
    var = qA / N - m * m
    rstd = lax.rsqrt(var + 1e-5)
    m4 = m[:, None]
    s4 = (pre_g * rstd)[:, None]
    b4 = pre_b[:, None]
    W0t = W0.T
    W1t = W1.T
    W2t = W2.T
    Wct = Wc.T

    momsB = _kb(fT, m4, s4, b4, W0t)
    m0 = momsB[:, 0] / N
    v0 = momsB[:, 1] / N - m0 * m0
    r0 = lax.rsqrt(v0 + 1e-5)
    s0 = (g0 * r0)[:, None]
    t0 = (b0 - m0 * g0 * r0)[:, None]

    momsC = _kc(fT, m4, s4, b4, W0t, s0, t0, W1t)
    m1 = momsC[:, 0] / N
    v1 = momsC[:, 1] / N - m1 * m1
    r1 = lax.rsqrt(v1 + 1e-5)
    s1 = (g1 * r1)[:, None]
    t1 = (b1 - m1 * g1 * r1)[:, None]

    hT = _kd(fT, m4, s4, b4, W0t, s0, t0, W1t, s1, t1, W2t, b2[:, None])

    ninf = jnp.full((NUM,), NEG_INF, jnp.float32)
    vmaxT = _seg(ids, hT.reshape(256 * N), ninf).reshape(256, NUM)

    vfT = _ke(vmaxT, Wct, bc[:, None])

    t5 = vfT.reshape(16, 16, 16, 16, 16)
    t = jnp.transpose(t5, (1, 0, 4, 3, 2))
    return jnp.pad(t, ((0, 0), (0, 0), (0, 0), (0, 48), (0, 48)))


# speedup vs baseline: 1.1023x; 1.0480x over previous
import functools

import jax
import jax.numpy as jnp
from jax import lax
from jax.experimental import pallas as pl
from jax.experimental.pallas import tpu as pltpu
from jax.experimental.pallas import tpu_sc as plsc

N = 160000
BN = 6400
NB = N // BN
NUM = 16 ** 4
CH = 16000
NCHUNK = N // CH
NWORK = 32
RPW = 256 // NWORK
BV = 4096
NBV = NUM // BV
NEG_INF = float("-inf")


def _hsw(x):
    return x * jnp.clip(x + 3.0, 0.0, 6.0) * (1.0 / 6.0)


def _dot(a, b):
    return lax.dot_general(a, b, (((1,), (0,)), ((), ())),
                           preferred_element_type=jnp.float32,
                           precision=lax.Precision.HIGHEST)


def _ka_body(ct_ref, ft_ref, ids_ref, moms_ref, acc_ref):
    i = pl.program_id(0)

    @pl.when(i == 0)
    def _():
        acc_ref[...] = jnp.zeros_like(acc_ref)

    c = ct_ref[...]
    ids_ref[...] = ((c[0:1] * 16 + c[1:2]) * 16 + c[2:3]) * 16 + c[3:4]
    f = ft_ref[...]
    acc_ref[0:4, 0:1] += jnp.sum(f, axis=1, keepdims=True)
    acc_ref[4:8, 0:1] += jnp.sum(f * f, axis=1, keepdims=True)
    moms_ref[...] = acc_ref[...]


_ka = pl.pallas_call(
    _ka_body,
    grid=(NB,),
    in_specs=[pl.BlockSpec((4, BN), lambda i: (0, i)),
              pl.BlockSpec((4, BN), lambda i: (0, i))],
    out_specs=[pl.BlockSpec((1, BN), lambda i: (0, i)),
               pl.BlockSpec((8, 128), lambda i: (0, 0))],
    out_shape=[jax.ShapeDtypeStruct((1, N), jnp.int32),
               jax.ShapeDtypeStruct((8, 128), jnp.float32)],
    scratch_shapes=[pltpu.VMEM((8, 128), jnp.float32)],
)


def _kb_body(ft_ref, m4_ref, s4_ref, b4_ref, w0_ref, moms_ref, acc_ref):
    i = pl.program_id(0)

    @pl.when(i == 0)
    def _():
        acc_ref[...] = jnp.zeros_like(acc_ref)

    x = (ft_ref[...] - m4_ref[...]) * s4_ref[...] + b4_ref[...]
    h0 = _dot(w0_ref[...], x)
    acc_ref[:, 0:1] += jnp.sum(h0, axis=1, keepdims=True)
    acc_ref[:, 1:2] += jnp.sum(h0 * h0, axis=1, keepdims=True)
    moms_ref[...] = acc_ref[...]


_kb = pl.pallas_call(
    _kb_body,
    grid=(NB,),
    in_specs=[pl.BlockSpec((4, BN), lambda i: (0, i)),
              pl.BlockSpec((4, 1), lambda i: (0, 0)),
              pl.BlockSpec((4, 1), lambda i: (0, 0)),
              pl.BlockSpec((4, 1), lambda i: (0, 0)),
              pl.BlockSpec((64, 4), lambda i: (0, 0))],
    out_specs=pl.BlockSpec((64, 128), lambda i: (0, 0)),
    out_shape=jax.ShapeDtypeStruct((64, 128), jnp.float32),
    scratch_shapes=[pltpu.VMEM((64, 128), jnp.float32)],
)


def _kc_body(ft_ref, m4_ref, s4_ref, b4_ref, w0_ref, s0_ref, t0_ref, w1_ref,
             moms_ref, acc_ref):
    i = pl.program_id(0)

    @pl.when(i == 0)
    def _():
        acc_ref[...] = jnp.zeros_like(acc_ref)

    x = (ft_ref[...] - m4_ref[...]) * s4_ref[...] + b4_ref[...]
    h0 = _dot(w0_ref[...], x)
    u = _hsw(h0 * s0_ref[...] + t0_ref[...])
    h1 = _dot(w1_ref[...], u)
    acc_ref[:, 0:1] += jnp.sum(h1, axis=1, keepdims=True)
    acc_ref[:, 1:2] += jnp.sum(h1 * h1, axis=1, keepdims=True)
    moms_ref[...] = acc_ref[...]


_kc = pl.pallas_call(
    _kc_body,
    grid=(NB,),
    in_specs=[pl.BlockSpec((4, BN), lambda i: (0, i)),
              pl.BlockSpec((4, 1), lambda i: (0, 0)),
              pl.BlockSpec((4, 1), lambda i: (0, 0)),
              pl.BlockSpec((4, 1), lambda i: (0, 0)),
              pl.BlockSpec((64, 4), lambda i: (0, 0)),
              pl.BlockSpec((64, 1), lambda i: (0, 0)),
              pl.BlockSpec((64, 1), lambda i: (0, 0)),
              pl.BlockSpec((128, 64), lambda i: (0, 0))],
    out_specs=pl.BlockSpec((128, 128), lambda i: (0, 0)),
    out_shape=jax.ShapeDtypeStruct((128, 128), jnp.float32),
    scratch_shapes=[pltpu.VMEM((128, 128), jnp.float32)],
)


def _kd_body(ft_ref, m4_ref, s4_ref, b4_ref, w0_ref, s0_ref, t0_ref, w1_ref,
             s1_ref, t1_ref, w2_ref, b2_ref, ht_ref):
    x = (ft_ref[...] - m4_ref[...]) * s4_ref[...] + b4_ref[...]
    h0 = _dot(w0_ref[...], x)
    u = _hsw(h0 * s0_ref[...] + t0_ref[...])
    h1 = _dot(w1_ref[...], u)
    v = _hsw(h1 * s1_ref[...] + t1_ref[...])
    ht_ref[...] = _dot(w2_ref[...], v) + b2_ref[...]


_kd = pl.pallas_call(
    _kd_body,
    grid=(NB,),
    in_specs=[pl.BlockSpec((4, BN), lambda i: (0, i)),
              pl.BlockSpec((4, 1), lambda i: (0, 0)),
              pl.BlockSpec((4, 1), lambda i: (0, 0)),
              pl.BlockSpec((4, 1), lambda i: (0, 0)),
              pl.BlockSpec((64, 4), lambda i: (0, 0)),
              pl.BlockSpec((64, 1), lambda i: (0, 0)),
              pl.BlockSpec((64, 1), lambda i: (0, 0)),
              pl.BlockSpec((128, 64), lambda i: (0, 0)),
              pl.BlockSpec((128, 1), lambda i: (0, 0)),
              pl.BlockSpec((128, 1), lambda i: (0, 0)),
              pl.BlockSpec((256, 128), lambda i: (0, 0)),
              pl.BlockSpec((256, 1), lambda i: (0, 0))],
    out_specs=pl.BlockSpec((256, BN), lambda i: (0, i)),
    out_shape=jax.ShapeDtypeStruct((256, N), jnp.float32),
    scratch_shapes=[],
)


def _perm16(x, idx):
    dn = lax.GatherDimensionNumbers(offset_dims=(), collapsed_slice_dims=(0,),
                                    start_index_map=(0,))
    return lax.gather(x, idx[:, None], dn, (1,),
                      mode=lax.GatherScatterMode.PROMISE_IN_BOUNDS)


def _seg_body(ids_hbm, ht_hbm, ninf_hbm, out_hbm, acc,
              idb0, valb0, idb1, valb1, si0, sv0, si1, sv1):
    cid = lax.axis_index("c")
    sid = lax.axis_index("s")
    wid = sid * 2 + cid
    bufs = ((idb0, valb0, si0, sv0), (idb1, valb1, si1, sv1))

    lanes = lax.iota(jnp.int32, 16)
    zero = jnp.zeros((16,), jnp.int32)
    sh1 = jnp.maximum(lanes - 1, zero)
    sh2 = jnp.maximum(lanes - 2, zero)
    sh4 = jnp.maximum(lanes - 4, zero)
    sh8 = jnp.maximum(lanes - 8, zero)
    dn1 = jnp.minimum(lanes + 1, zero + 15)
    is15 = lanes == (zero + 15)

    def round_body(r, carry):
        ch = wid * RPW + r
        pltpu.sync_copy(ninf_hbm, acc)

        def start(k, idb, valb, semi, semv):
            base = lax.rem(k + wid, NCHUNK) * CH
            ci = pltpu.async_copy(ids_hbm.at[pl.ds(base, CH)], idb, semi)
            cv = pltpu.async_copy(ht_hbm.at[pl.ds(ch * N + base, CH)],
                                  valb, semv)
            return ci, cv

        def process(idb, valb):
            def vec_body(j, carry3):
                idv = idb[pl.ds(j * 16, 16)]
                val = valb[pl.ds(j * 16, 16)]
                sk, sv = plsc.sort_key_val(idv, val)
                for sh in (sh1, sh2, sh4, sh8):
                    pk = _perm16(sk, sh)
                    pv = _perm16(sv, sh)
                    sv = jnp.maximum(sv, jnp.where(pk == sk, pv, NEG_INF))
                last = (sk != _perm16(sk, dn1)) | is15
                cur = plsc.load_gather(acc, [sk])
                m = last & (sv > cur)
                plsc.store_scatter(acc, [sk], sv, mask=m)
                return carry3

            lax.fori_loop(0, CH // 16, vec_body, 0)

        pend = start(0, *bufs[0])
        for k in range(NCHUNK):
            idb, valb = bufs[k % 2][0], bufs[k % 2][1]
            nxt = None
            if k + 1 < NCHUNK:
                nxt = start(k + 1, *bufs[(k + 1) % 2])
            pend[0].wait()
            pend[1].wait()
            process(idb, valb)
            pend = nxt
        pltpu.sync_copy(acc, out_hbm.at[pl.ds(ch * NUM, NUM)])
        return carry

    lax.fori_loop(0, RPW, round_body, 0)


_seg = pl.kernel(
    _seg_body,
    out_type=jax.ShapeDtypeStruct((256 * NUM,), jnp.float32),
    scratch_types=[pltpu.VMEM((NUM,), jnp.float32),
                   pltpu.VMEM((CH,), jnp.int32),
                   pltpu.VMEM((CH,), jnp.float32),
                   pltpu.VMEM((CH,), jnp.int32),
                   pltpu.VMEM((CH,), jnp.float32),
                   pltpu.SemaphoreType.DMA,
                   pltpu.SemaphoreType.DMA,
                   pltpu.SemaphoreType.DMA,
                   pltpu.SemaphoreType.DMA],
    mesh=plsc.VectorSubcoreMesh(core_axis_name="c", subcore_axis_name="s"),
    compiler_params=pltpu.CompilerParams(needs_layout_passes=False),
)


def _ke_body(vm_ref, wc_ref, bc_ref, vf_ref):
    vb = vm_ref[...]
    occ = vb[0:1, :] > NEG_INF
    vm = jnp.where(occ, vb, 0.0)
    z = _dot(wc_ref[...], vm) + bc_ref[...]
    vf_ref[...] = jnp.where(occ, _hsw(z), 0.0)


_ke = pl.pallas_call(
    _ke_body,
    grid=(NBV,),
    in_specs=[pl.BlockSpec((256, BV), lambda i: (0, i)),
              pl.BlockSpec((16, 256), lambda i: (0, 0)),
              pl.BlockSpec((16, 1), lambda i: (0, 0))],
    out_specs=pl.BlockSpec((16, BV), lambda i: (0, i)),
    out_shape=jax.ShapeDtypeStruct((16, NUM), jnp.float32),
    scratch_shapes=[],
)


def kernel(features, coors, pre_g, pre_b, W0, g0, b0, W1, g1, b1, W2, b2, Wc, bc):
    fT = features.T
    cT = coors.T
    ids2d, momsA = _ka(cT, fT)
    ids = ids2d.reshape(N)

    sA = momsA[0:4, 0]
    qA = momsA[4:8, 0]
    m = sA / N
    var = qA / N - m * m
    rstd = lax.rsqrt(var + 1e-5)
    m4 = m[:, None]
    s4 = (pre_g * rstd)[:, None]
    b4 = pre_b[:, None]
    W0t = W0.T
    W1t = W1.T
    W2t = W2.T
    Wct = Wc.T

    momsB = _kb(fT, m4, s4, b4, W0t)
    m0 = momsB[:, 0] / N
    v0 = momsB[:, 1] / N - m0 * m0
    r0 = lax.rsqrt(v0 + 1e-5)
    s0 = (g0 * r0)[:, None]
    t0 = (b0 - m0 * g0 * r0)[:, None]

    momsC = _kc(fT, m4, s4, b4, W0t, s0, t0, W1t)
    m1 = momsC[:, 0] / N
    v1 = momsC[:, 1] / N - m1 * m1
    r1 = lax.rsqrt(v1 + 1e-5)
    s1 = (g1 * r1)[:, None]
    t1 = (b1 - m1 * g1 * r1)[:, None]

    hT = _kd(fT, m4, s4, b4, W0t, s0, t0, W1t, s1, t1, W2t, b2[:, None])

    ninf = jnp.full((NUM,), NEG_INF, jnp.float32)
    vmaxT = _seg(ids, hT.reshape(256 * N), ninf).reshape(256, NUM)

    vfT = _ke(vmaxT, Wct, bc[:, None])

    t5 = vfT.reshape(16, 16, 16, 16, 16)
    t = jnp.transpose(t5, (1, 0, 4, 3, 2))
    return jnp.pad(t, ((0, 0), (0, 0), (0, 0), (0, 48), (0, 48)))
